# baseline (speedup 1.0000x reference)
import functools

import jax
import jax.numpy as jnp
from jax.experimental import pallas as pl
from jax.experimental.pallas import tpu as pltpu

BETA = 2.0
LOSS_WEIGHT = 1.0


def _qfl_block(pred_ref, vref, out_ref, acc_ref):
    i = pl.program_id(0)
    nb = pl.num_programs(0)

    @pl.when(i == 0)
    def _init():
        acc_ref[...] = jnp.zeros_like(acc_ref)

    p = pred_ref[...]
    B, C = p.shape

    e = jnp.exp(-p)
    u = 1.0 + e
    sp = p + jnp.log(u)
    d = 1.0 / u

    v = vref[0].reshape(B, 1)
    two_col = 2.0 * jax.lax.broadcasted_iota(jnp.int32, (B, C), 1).astype(jnp.float32)
    h = v - two_col
    q = jnp.where((h >= 0.0) & (h < 1.0), h, 0.0)

    t = q - d
    contrib = (sp - q * p) * (t * t)
    acc_ref[...] += jnp.sum(contrib.reshape(B // 8, 8, C), axis=0)

    @pl.when(i == nb - 1)
    def _fin():
        out_ref[...] = jnp.full((1, 1), jnp.sum(acc_ref[...]), jnp.float32)


@functools.partial(jax.jit, static_argnames=("block_rows",))
def _qfl(pred, label, score, block_rows=10000):
    N, C = pred.shape
    nb = N // block_rows
    v3 = (2.0 * label.astype(jnp.float32) + score).reshape(nb, 1, block_rows)

    total = pl.pallas_call(
        _qfl_block,
        grid=(nb,),
        in_specs=[
            pl.BlockSpec((block_rows, C), lambda i: (i, 0)),
            pl.BlockSpec((1, 1, block_rows), lambda i: (i, 0, 0)),
        ],
        out_specs=pl.BlockSpec((1, 1), lambda i: (0, 0)),
        out_shape=jax.ShapeDtypeStruct((1, 1), jnp.float32),
        scratch_shapes=[pltpu.VMEM((8, C), jnp.float32)],
    )(pred, v3)

    return LOSS_WEIGHT * total[0, 0] / N


def kernel(pred, label, score):
    return _qfl(pred, label, score)


# speedup vs baseline: 1.0057x; 1.0057x over previous
import functools

import jax
import jax.numpy as jnp
from jax.experimental import pallas as pl
from jax.experimental.pallas import tpu as pltpu

BETA = 2.0
LOSS_WEIGHT = 1.0


def _qfl_block(pred_ref, vref, out_ref, acc_ref):
    i = pl.program_id(0)
    nb = pl.num_programs(0)

    @pl.when(i == 0)
    def _init():
        acc_ref[...] = jnp.zeros_like(acc_ref)

    p = pred_ref[...]
    B, C = p.shape

    e = jnp.exp(-p)
    u = 1.0 + e
    sp = p + jnp.log(u)
    d = 1.0 / u

    v = vref[0].reshape(B, 1)
    two_col = 2.0 * jax.lax.broadcasted_iota(jnp.int32, (B, C), 1).astype(jnp.float32)
    h = v - two_col
    q = jnp.where((h >= 0.0) & (h < 1.0), h, 0.0)

    t = q - d
    contrib = (sp - q * p) * (t * t)
    acc_ref[...] += jnp.sum(contrib.reshape(B // 8, 8, C), axis=0)

    @pl.when(i == nb - 1)
    def _fin():
        out_ref[...] = jnp.full((1, 1), jnp.sum(acc_ref[...]), jnp.float32)


@functools.partial(jax.jit, static_argnames=("block_rows",))
def _qfl(pred, label, score, block_rows=5000):
    N, C = pred.shape
    nb = N // block_rows
    v3 = (2.0 * label.astype(jnp.float32) + score).reshape(nb, 1, block_rows)

    total = pl.pallas_call(
        _qfl_block,
        grid=(nb,),
        in_specs=[
            pl.BlockSpec((block_rows, C), lambda i: (i, 0)),
            pl.BlockSpec((1, 1, block_rows), lambda i: (i, 0, 0)),
        ],
        out_specs=pl.BlockSpec((1, 1), lambda i: (0, 0)),
        out_shape=jax.ShapeDtypeStruct((1, 1), jnp.float32),
        scratch_shapes=[pltpu.VMEM((8, C), jnp.float32)],
    )(pred, v3)

    return LOSS_WEIGHT * total[0, 0] / N


def kernel(pred, label, score):
    return _qfl(pred, label, score)
